# baseline (speedup 1.0000x reference)
import functools

import jax
import jax.numpy as jnp
from jax import lax
from jax.experimental import pallas as pl
from jax.experimental.pallas import tpu as pltpu
from jax.experimental.pallas import tpu_sc as plsc

_INFO = plsc.get_sparse_core_info()
_NC = _INFO.num_cores
_NS = _INFO.num_subcores
_NW = _NC * _NS

_CHUNK = 128
_NBUF = 4


def _gather_grid(total, dim, chunks_per_w):
    mesh = plsc.VectorSubcoreMesh(core_axis_name="c", subcore_axis_name="s")

    @functools.partial(
        pl.kernel,
        mesh=mesh,
        compiler_params=pltpu.CompilerParams(use_tc_tiling_on_sc=False),
        out_type=jax.ShapeDtypeStruct((total, dim), jnp.float32),
        scratch_types=[
            pltpu.VMEM((chunks_per_w, _CHUNK), jnp.int32),
            pltpu.VMEM((_NBUF, _CHUNK, dim), jnp.float32),
            pltpu.SemaphoreType.DMA,
        ],
    )
    def body(ids_hbm, table_hbm, out_hbm, idx_v, rows_v, gsem):
        wid = lax.axis_index("s") * _NC + lax.axis_index("c")
        row0 = wid * chunks_per_w
        pltpu.sync_copy(ids_hbm.at[pl.ds(row0, chunks_per_w)], idx_v)

        @pl.loop(0, chunks_per_w, step=_NBUF)
        def _(g):
            copies = []
            for b in range(_NBUF):
                j = g + b
                copies.append(
                    pltpu.async_copy(table_hbm.at[idx_v.at[j]], rows_v.at[b], gsem))
            for b in range(_NBUF):
                j = g + b
                copies[b].wait()
                pltpu.sync_copy(rows_v.at[b],
                                out_hbm.at[pl.ds((row0 + j) * _CHUNK, _CHUNK)])

    return body


def kernel(node_ids, embeddings):
    batch, hist = node_ids.shape
    _, dim = embeddings.shape
    total = batch * hist
    per_w = total // _NW
    chunks_per_w = per_w // _CHUNK

    ids2d = node_ids.reshape(total // _CHUNK, _CHUNK)
    out = _gather_grid(total, dim, chunks_per_w)(ids2d, embeddings)
    return out.reshape(batch, hist, dim)


# speedup vs baseline: 1.0082x; 1.0082x over previous
import functools

import jax
import jax.numpy as jnp
from jax import lax
from jax.experimental import pallas as pl
from jax.experimental.pallas import tpu as pltpu
from jax.experimental.pallas import tpu_sc as plsc

_INFO = plsc.get_sparse_core_info()
_NC = _INFO.num_cores
_NS = _INFO.num_subcores
_NW = _NC * _NS

_CHUNK = 128
_NBUF = 8


def _gather_grid(total, dim, chunks_per_w):
    mesh = plsc.VectorSubcoreMesh(core_axis_name="c", subcore_axis_name="s")

    @functools.partial(
        pl.kernel,
        mesh=mesh,
        compiler_params=pltpu.CompilerParams(use_tc_tiling_on_sc=False),
        out_type=jax.ShapeDtypeStruct((total, dim), jnp.float32),
        scratch_types=[
            pltpu.VMEM((chunks_per_w, _CHUNK), jnp.int32),
            pltpu.VMEM((_NBUF, _CHUNK, dim), jnp.float32),
            pltpu.SemaphoreType.DMA,
            pltpu.SemaphoreType.DMA,
        ],
    )
    def body(ids_hbm, table_hbm, out_hbm, idx_v, rows_v, gsem, ssem):
        wid = lax.axis_index("s") * _NC + lax.axis_index("c")
        row0 = wid * chunks_per_w

        def gather(j, b):
            pltpu.async_copy(table_hbm.at[idx_v.at[j]], rows_v.at[b], gsem)

        def gather_wait(b):
            pltpu.make_async_copy(table_hbm.at[idx_v.at[0]], rows_v.at[b],
                                  gsem).wait()

        def store(j, b):
            pltpu.async_copy(rows_v.at[b],
                             out_hbm.at[pl.ds((row0 + j) * _CHUNK, _CHUNK)], ssem)

        def store_wait(j, b):
            pltpu.make_async_copy(rows_v.at[b],
                                  out_hbm.at[pl.ds((row0 + j) * _CHUNK, _CHUNK)],
                                  ssem).wait()

        pltpu.sync_copy(ids_hbm.at[pl.ds(row0, chunks_per_w)], idx_v)
        for b in range(_NBUF):
            gather(b, b)

        @pl.loop(0, (chunks_per_w - _NBUF) // _NBUF)
        def _(t):
            g = t * _NBUF
            for b in range(_NBUF):
                gather_wait(b)
                store(g + b, b)
            for b in range(_NBUF):
                store_wait(g + b, b)
                gather(g + _NBUF + b, b)

        gl = chunks_per_w - _NBUF
        for b in range(_NBUF):
            gather_wait(b)
            store(gl + b, b)
        for b in range(_NBUF):
            store_wait(gl + b, b)

    return body


def kernel(node_ids, embeddings):
    batch, hist = node_ids.shape
    _, dim = embeddings.shape
    total = batch * hist
    per_w = total // _NW
    chunks_per_w = per_w // _CHUNK

    ids2d = node_ids.reshape(total // _CHUNK, _CHUNK)
    out = _gather_grid(total, dim, chunks_per_w)(ids2d, embeddings)
    return out.reshape(batch, hist, dim)


# speedup vs baseline: 1.0341x; 1.0257x over previous
import functools

import jax
import jax.numpy as jnp
from jax import lax
from jax.experimental import pallas as pl
from jax.experimental.pallas import tpu as pltpu
from jax.experimental.pallas import tpu_sc as plsc

_INFO = plsc.get_sparse_core_info()
_NC = _INFO.num_cores
_NS = _INFO.num_subcores
_NW = _NC * _NS

_CHUNK = 128
_NBUF = 8


def _gather_grid(total, dim, chunks_per_w):
    mesh = plsc.VectorSubcoreMesh(core_axis_name="c", subcore_axis_name="s")

    @functools.partial(
        pl.kernel,
        mesh=mesh,
        compiler_params=pltpu.CompilerParams(use_tc_tiling_on_sc=False),
        out_type=jax.ShapeDtypeStruct((total, dim), jnp.float32),
        scratch_types=[
            pltpu.VMEM((chunks_per_w, _CHUNK), jnp.int32),
            pltpu.VMEM((_NBUF, _CHUNK, dim), jnp.float32),
            pltpu.SemaphoreType.DMA,
            pltpu.SemaphoreType.DMA,
        ],
    )
    def body(ids_hbm, table_hbm, out_hbm, idx_v, rows_v, gsem, ssem):
        wid = lax.axis_index("s") * _NC + lax.axis_index("c")
        row0 = wid * chunks_per_w

        def gather(j, b):
            pltpu.async_copy(table_hbm.at[idx_v.at[j]], rows_v.at[b], gsem)

        def gather_wait(b):
            pltpu.make_async_copy(table_hbm.at[idx_v.at[0]], rows_v.at[b],
                                  gsem).wait()

        def store(j, b):
            pltpu.async_copy(rows_v.at[b],
                             out_hbm.at[pl.ds((row0 + j) * _CHUNK, _CHUNK)], ssem)

        def store_wait(j, b):
            pltpu.make_async_copy(rows_v.at[b],
                                  out_hbm.at[pl.ds((row0 + j) * _CHUNK, _CHUNK)],
                                  ssem).wait()

        pltpu.sync_copy(ids_hbm.at[pl.ds(row0, chunks_per_w)], idx_v)
        for b in range(_NBUF):
            gather(b, b)

        @pl.loop(0, (chunks_per_w - _NBUF) // _NBUF)
        def _(t):
            g = t * _NBUF
            for b in range(_NBUF):
                gather_wait(b)
                store(g + b, b)
            for b in range(_NBUF):
                store_wait(g + b, b)
                gather(g + _NBUF + b, b)

        gl = chunks_per_w - _NBUF
        for b in range(_NBUF):
            gather_wait(b)
            store(gl + b, b)
        for b in range(_NBUF):
            store_wait(gl + b, b)

    return body


def kernel(node_ids, embeddings):
    batch, hist = node_ids.shape
    _, dim = embeddings.shape
    total = batch * hist
    per_w = total // _NW
    chunks_per_w = per_w // _CHUNK

    ids2d = node_ids.T.reshape(total // _CHUNK, _CHUNK)
    out = _gather_grid(total, dim, chunks_per_w)(ids2d, embeddings)
    return out.reshape(hist, batch, dim).transpose(1, 0, 2)
